# baseline (speedup 1.0000x reference)
import functools

import jax
import jax.numpy as jnp
from jax import lax
from jax.experimental import pallas as pl
from jax.experimental.pallas import tpu as pltpu
from jax.experimental.pallas import tpu_sc as plsc

CHUNK = 128


@functools.partial(jax.jit, static_argnames=("n_chunks", "dim"))
def _sc_gather(weight, idx3, n_chunks, dim):
    nw = idx3.shape[0]
    per_tile = n_chunks * CHUNK
    mesh = plsc.VectorSubcoreMesh(core_axis_name="c", subcore_axis_name="s")

    @functools.partial(
        pl.kernel,
        mesh=mesh,
        out_type=jax.ShapeDtypeStruct((nw * per_tile, dim), jnp.float32),
        scratch_types=[
            pltpu.VMEM((n_chunks, CHUNK), jnp.int32),
            pltpu.VMEM((CHUNK, dim), jnp.float32),
            pltpu.SemaphoreType.DMA,
        ],
        compiler_params=pltpu.CompilerParams(use_tc_tiling_on_sc=False),
    )
    def k(weight_hbm, idx_hbm, out_hbm, idx_v, rows, sem):
        wid = lax.axis_index("s") * 2 + lax.axis_index("c")
        base = wid * per_tile
        pltpu.sync_copy(idx_hbm.at[wid], idx_v)

        def body(j, carry):
            pltpu.async_copy(weight_hbm.at[idx_v.at[j]], rows, sem).wait()
            pltpu.sync_copy(rows, out_hbm.at[pl.ds(base + j * CHUNK, CHUNK)])
            return carry

        lax.fori_loop(0, n_chunks, body, 0)

    return k(weight, idx3)


def kernel(x, weight):
    b, f = x.shape
    dim = weight.shape[1]
    n = b * f
    nw = 32
    assert n % (nw * CHUNK) == 0
    n_chunks = n // (nw * CHUNK)
    idx3 = x.reshape(nw, n_chunks, CHUNK).astype(jnp.int32)
    out = _sc_gather(weight, idx3, n_chunks, dim)
    return out.reshape(b, f, dim)


# speedup vs baseline: 1.0790x; 1.0790x over previous
import functools

import jax
import jax.numpy as jnp
from jax import lax
from jax.experimental import pallas as pl
from jax.experimental.pallas import tpu as pltpu
from jax.experimental.pallas import tpu_sc as plsc

CHUNK = 128
NBUF = 4


@functools.partial(jax.jit, static_argnames=("n_chunks", "dim"))
def _sc_gather(weight, idx3, n_chunks, dim):
    nw = idx3.shape[0]
    per_tile = n_chunks * CHUNK
    mesh = plsc.VectorSubcoreMesh(core_axis_name="c", subcore_axis_name="s")

    @functools.partial(
        pl.kernel,
        mesh=mesh,
        out_type=jax.ShapeDtypeStruct((nw * per_tile, dim), jnp.float32),
        scratch_types=[
            pltpu.VMEM((n_chunks, CHUNK), jnp.int32),
            pltpu.VMEM((NBUF, CHUNK, dim), jnp.float32),
            pltpu.SemaphoreType.DMA,
            pltpu.SemaphoreType.DMA,
            pltpu.SemaphoreType.DMA,
            pltpu.SemaphoreType.DMA,
            pltpu.SemaphoreType.DMA,
            pltpu.SemaphoreType.DMA,
            pltpu.SemaphoreType.DMA,
            pltpu.SemaphoreType.DMA,
        ],
        compiler_params=pltpu.CompilerParams(use_tc_tiling_on_sc=False),
    )
    def k(weight_hbm, idx_hbm, out_hbm, idx_v, rows, g0, g1, g2, g3, w0, w1, w2, w3):
        gsems = (g0, g1, g2, g3)
        wsems = (w0, w1, w2, w3)
        n_groups = n_chunks // NBUF
        wid = lax.axis_index("s") * 2 + lax.axis_index("c")
        base = wid * per_tile
        pltpu.sync_copy(idx_hbm.at[wid], idx_v)

        for b in range(NBUF):
            pltpu.async_copy(weight_hbm.at[idx_v.at[b]], rows.at[b], gsems[b])

        def body(p, carry):
            for b in range(NBUF):
                j = p * NBUF + b
                dst = out_hbm.at[pl.ds(base + j * CHUNK, CHUNK)]
                pltpu.make_async_copy(
                    weight_hbm.at[idx_v.at[j]], rows.at[b], gsems[b]
                ).wait()
                pltpu.async_copy(rows.at[b], dst, wsems[b])

                @pl.when(p < n_groups - 1)
                def _(b=b, j=j, dst=dst):
                    pltpu.make_async_copy(rows.at[b], dst, wsems[b]).wait()
                    pltpu.async_copy(
                        weight_hbm.at[idx_v.at[j + NBUF]], rows.at[b], gsems[b]
                    )

            return carry

        lax.fori_loop(0, n_groups, body, 0)

        for b in range(NBUF):
            j = n_chunks - NBUF + b
            pltpu.make_async_copy(
                rows.at[b], out_hbm.at[pl.ds(base + j * CHUNK, CHUNK)], wsems[b]
            ).wait()

    return k(weight, idx3)


def kernel(x, weight):
    b, f = x.shape
    dim = weight.shape[1]
    n = b * f
    nw = 32
    assert n % (nw * CHUNK) == 0
    n_chunks = n // (nw * CHUNK)
    idx3 = x.reshape(nw, n_chunks, CHUNK).astype(jnp.int32)
    out = _sc_gather(weight, idx3, n_chunks, dim)
    return out.reshape(b, f, dim)
